# baseline (speedup 1.0000x reference)
import functools

import jax
import jax.numpy as jnp
from jax import lax
from jax.experimental import pallas as pl
from jax.experimental.pallas import tpu as pltpu
from jax.experimental.pallas import tpu_sc as plsc

N_USERS = 30000
N_ITEMS = 20000
N_NODES = N_USERS + N_ITEMS
D = 64
E = 800000
EH = E // 2
LAYERS = 3

NC = 2
NS = 16
LANES = 16

PAD_ROWS = 80
ITEM_BASE = N_USERS + PAD_ROWS
N_PAD = N_USERS + PAD_ROWS + N_ITEMS

CHUNK = 128
NBR = 3
NBI = 6
CHUNKS_PER_TILE = 396
EDGES_PER_TILE = CHUNK * CHUNKS_PER_TILE
E_PAD = EDGES_PER_TILE * NS

U_OWN = N_USERS // 2
I_OWN = N_ITEMS // 2
ACC_ITEM = U_OWN
DUMMY = U_OWN + I_OWN
ACC_ROWS = 25088
ZROWS = ACC_ROWS // NS

LOOK_I = 4
LOOK_G = 2
LAG_S = 1

U_RPT, U_LAST = 936, U_OWN - 15 * 936
I_RPT, I_LAST = 624, I_OWN - 15 * 624

_mesh = plsc.VectorSubcoreMesh(core_axis_name="c", subcore_axis_name="s")

_BCAST_DNUMS = lax.GatherDimensionNumbers(
    offset_dims=(), collapsed_slice_dims=(0,), start_index_map=(0,)
)


def _lane_bcast(v, lane_idx):
    idx = jnp.full((LANES, 1), lane_idx, jnp.int32)
    return lax.gather(
        v, idx, _BCAST_DNUMS, (1,), mode=lax.GatherScatterMode.PROMISE_IN_BOUNDS
    )
_params = pltpu.CompilerParams(needs_layout_passes=False, use_tc_tiling_on_sc=False)

_prop_scratch = (
    [
        pltpu.VMEM_SHARED((ACC_ROWS, D), jnp.float32),
        pltpu.VMEM((NBI, CHUNK), jnp.int32),
        pltpu.VMEM((NBI, CHUNK), jnp.float32),
    ]
    + [pltpu.VMEM((CHUNK,), jnp.int32) for _ in range(NBI)]
    + [pltpu.VMEM((CHUNK, D), jnp.float32) for _ in range(NBR)]
    + [pltpu.SemaphoreType.DMA for _ in range(NBI + 2 * NBR)]
)


@functools.partial(
    pl.kernel,
    mesh=_mesh,
    compiler_params=_params,
    out_type=jax.ShapeDtypeStruct((N_PAD, D), jnp.float32),
    scratch_types=_prop_scratch,
)
def _propagate(emb, srcp, dstp, wp, zrows, out, acc, src_r, w_r, *rest):
    dst_r = rest[0:NBI]
    rows = rest[NBI : NBI + NBR]
    sem_i = rest[NBI + NBR : 2 * NBI + NBR]
    sem_g = rest[2 * NBI + NBR : 2 * NBI + 2 * NBR]
    sem_s = rest[2 * NBI + 2 * NBR : 2 * NBI + 3 * NBR]

    c = lax.axis_index("c")
    s = lax.axis_index("s")
    ku = c * U_OWN
    ki = c * I_OWN
    edge_base = s * EDGES_PER_TILE

    pltpu.sync_copy(zrows, acc.at[pl.ds(s * ZROWS, ZROWS)])
    plsc.subcore_barrier()

    def fire_idx(q, slot):
        base = edge_base + q * CHUNK
        pltpu.async_copy(srcp.at[pl.ds(base, CHUNK)], src_r.at[slot], sem_i[slot])
        pltpu.async_copy(dstp.at[pl.ds(base, CHUNK)], dst_r[slot], sem_i[slot])
        pltpu.async_copy(wp.at[pl.ds(base, CHUNK)], w_r.at[slot], sem_i[slot])

    def wait_idx(q, slot):
        base = edge_base + q * CHUNK
        pltpu.make_async_copy(srcp.at[pl.ds(base, CHUNK)], src_r.at[slot], sem_i[slot]).wait()
        pltpu.make_async_copy(dstp.at[pl.ds(base, CHUNK)], dst_r[slot], sem_i[slot]).wait()
        pltpu.make_async_copy(wp.at[pl.ds(base, CHUNK)], w_r.at[slot], sem_i[slot]).wait()

    def fire_gather(islot, rslot):
        pltpu.async_copy(emb.at[src_r.at[islot]], rows[rslot], sem_g[rslot])

    def wait_gather(islot, rslot):
        pltpu.make_async_copy(emb.at[src_r.at[islot]], rows[rslot], sem_g[rslot]).wait()

    def fire_scatter(islot, rslot):
        pltpu.async_copy(rows[rslot], acc.at[dst_r[islot]], sem_s[rslot], add=True)

    def wait_scatter(islot, rslot):
        pltpu.make_async_copy(rows[rslot], acc.at[dst_r[islot]], sem_s[rslot]).wait()

    for q0 in range(LOOK_I):
        fire_idx(q0, q0)
    for q0 in range(LOOK_G):
        wait_idx(q0, q0)
        fire_gather(q0, q0)

    def outer(sg, carry):
        q0 = sg * NBI
        for u in range(NBI):
            q = q0 + u
            b = u % NBR

            @pl.when(q >= LAG_S)
            def _(u=u, b=b):
                wait_scatter((u - LAG_S) % NBI, (b - LAG_S) % NBR)

            @pl.when(q + LOOK_I < CHUNKS_PER_TILE)
            def _(q=q, u=u):
                fire_idx(q + LOOK_I, (u + LOOK_I) % NBI)

            @pl.when(q + LOOK_G < CHUNKS_PER_TILE)
            def _(q=q, u=u, b=b):
                wait_idx(q + LOOK_G, (u + LOOK_G) % NBI)
                fire_gather((u + LOOK_G) % NBI, (b + LOOK_G) % NBR)

            wait_gather(u, b)

            def grp(gg, carry2, u=u, b=b):
                o = gg * LANES
                d = dst_r[u][pl.ds(o, LANES)]
                isu = d < N_USERS
                lu = d - ku
                li = d - (N_USERS + ki) + ACC_ITEM
                loc = jnp.where(isu, lu, li)
                ok = jnp.where(
                    isu,
                    (lu >= 0) & (lu < U_OWN),
                    (li >= ACC_ITEM) & (li < DUMMY),
                )
                dummy = DUMMY + ((d + gg) & 63)
                dst_r[u][pl.ds(o, LANES)] = jnp.where(ok, loc, dummy)
                wv = w_r[u, pl.ds(o, LANES)]
                wes = [_lane_bcast(wv, e16) for e16 in range(LANES)]
                nj = D // LANES
                for blk in range(0, LANES, 4):
                    es = [o + blk + t for t in range(4)]
                    vals = [
                        rows[b][e, pl.ds(j * LANES, LANES)]
                        for e in es
                        for j in range(nj)
                    ]
                    scaled = [
                        vals[t * nj + j] * wes[blk + t]
                        for t in range(4)
                        for j in range(nj)
                    ]
                    for t in range(4):
                        for j in range(nj):
                            rows[b][es[t], pl.ds(j * LANES, LANES)] = scaled[t * nj + j]
                return carry2

            lax.fori_loop(0, CHUNK // LANES, grp, 0)
            fire_scatter(u, b)
        return carry

    lax.fori_loop(0, CHUNKS_PER_TILE // NBI, outer, 0)
    for qq in range(CHUNKS_PER_TILE - LAG_S, CHUNKS_PER_TILE):
        wait_scatter(qq % NBI, qq % NBR)

    plsc.subcore_barrier()

    @pl.when(s < 15)
    def _():
        pltpu.sync_copy(
            acc.at[pl.ds(s * U_RPT, U_RPT)], out.at[pl.ds(c * U_OWN + s * U_RPT, U_RPT)]
        )
        pltpu.sync_copy(
            acc.at[pl.ds(ACC_ITEM + s * I_RPT, I_RPT)],
            out.at[pl.ds(ITEM_BASE + c * I_OWN + s * I_RPT, I_RPT)],
        )

    @pl.when(s == 15)
    def _():
        pltpu.sync_copy(
            acc.at[pl.ds(15 * U_RPT, U_LAST)],
            out.at[pl.ds(c * U_OWN + 15 * U_RPT, U_LAST)],
        )
        pltpu.sync_copy(
            acc.at[pl.ds(ACC_ITEM + 15 * I_RPT, I_LAST)],
            out.at[pl.ds(ITEM_BASE + c * I_OWN + 15 * I_RPT, I_LAST)],
        )


def _make_forward(batch):
    bt = batch // (NC * NS)

    @functools.partial(
        pl.kernel,
        mesh=_mesh,
        compiler_params=_params,
        out_type=jax.ShapeDtypeStruct((batch,), jnp.float32),
        scratch_types=[
            pltpu.VMEM((bt,), jnp.int32),
            pltpu.VMEM((bt,), jnp.int32),
            pltpu.VMEM((bt, D), jnp.float32),
            pltpu.VMEM((bt, D), jnp.float32),
            pltpu.VMEM((bt, D), jnp.float32),
            pltpu.VMEM((bt,), jnp.float32),
            pltpu.SemaphoreType.DMA,
        ],
    )
    def _forward(e0, e1, e2, e3, users, items, gamma, uidx, iidx, rows, uacc, iacc, gam, sem):
        c = lax.axis_index("c")
        s = lax.axis_index("s")
        b0 = (s * NC + c) * bt
        pltpu.sync_copy(users.at[pl.ds(b0, bt)], uidx)
        pltpu.sync_copy(items.at[pl.ds(b0, bt)], iidx)

        def off_body(gg, carry):
            o = gg * LANES
            iidx[pl.ds(o, LANES)] = iidx[pl.ds(o, LANES)] + ITEM_BASE
            return carry

        lax.fori_loop(0, bt // LANES, off_body, 0)

        for idx, dacc in ((uidx, uacc), (iidx, iacc)):
            for t, tab in enumerate((e0, e1, e2, e3)):
                pltpu.async_copy(tab.at[idx], rows, sem).wait()

                def acc_body(e, carry, t=t, dacc=dacc):
                    for j in range(D // LANES):
                        sl = pl.ds(j * LANES, LANES)
                        v = rows[e, sl]
                        if t:
                            v = dacc[e, sl] + v
                        dacc[e, sl] = v
                    return carry

                lax.fori_loop(0, bt, acc_body, 0)

        scale = 1.0 / float((LAYERS + 1) ** 2)
        lane = lax.iota(jnp.int32, LANES)

        def dot_body(g, carry):
            o = g * LANES
            accv = jnp.zeros((LANES,), jnp.float32)
            for e16 in range(LANES):
                e = o + e16
                ps = jnp.zeros((LANES,), jnp.float32)
                for j in range(D // LANES):
                    sl = pl.ds(j * LANES, LANES)
                    ps = ps + uacc[e, sl] * iacc[e, sl]
                tot = jnp.sum(ps) * scale
                accv = jnp.where(lane == e16, tot, accv)
            gam[pl.ds(o, LANES)] = accv
            return carry

        lax.fori_loop(0, bt // LANES, dot_body, 0)
        pltpu.sync_copy(gam, gamma.at[pl.ds(b0, bt)])

    return _forward


def kernel(users, items, edge_index, edge_weight, user_table, item_table):
    emb0 = jnp.concatenate(
        [user_table, jnp.zeros((PAD_ROWS, D), jnp.float32), item_table], axis=0
    )
    src = edge_index[0]
    dst = edge_index[1]
    src = jnp.where(src >= N_USERS, src + PAD_ROWS, src)
    pad = E_PAD - E
    srcp = jnp.concatenate([src, jnp.zeros((pad,), jnp.int32)])
    dstp = jnp.concatenate([dst, jnp.zeros((pad,), jnp.int32)])
    wp = jnp.concatenate([edge_weight, jnp.zeros((pad,), jnp.float32)])
    zrows = jnp.zeros((ZROWS, D), jnp.float32)

    e0 = emb0
    e1 = _propagate(e0, srcp, dstp, wp, zrows)
    e2 = _propagate(e1, srcp, dstp, wp, zrows)
    e3 = _propagate(e2, srcp, dstp, wp, zrows)
    fwd = _make_forward(users.shape[0])
    return fwd(e0, e1, e2, e3, users, items)


# speedup vs baseline: 1.0190x; 1.0190x over previous
import functools

import jax
import jax.numpy as jnp
from jax import lax
from jax.experimental import pallas as pl
from jax.experimental.pallas import tpu as pltpu
from jax.experimental.pallas import tpu_sc as plsc

N_USERS = 30000
N_ITEMS = 20000
N_NODES = N_USERS + N_ITEMS
D = 64
E = 800000
EH = E // 2
LAYERS = 3

NC = 2
NS = 16
LANES = 16

PAD_ROWS = 80
ITEM_BASE = N_USERS + PAD_ROWS
N_PAD = N_USERS + PAD_ROWS + N_ITEMS

CHUNK = 128
NBR = 3
NBI = 6
CHUNKS_PER_TILE = 396
EDGES_PER_TILE = CHUNK * CHUNKS_PER_TILE
E_PAD = EDGES_PER_TILE * NS

U_OWN = N_USERS // 2
I_OWN = N_ITEMS // 2
ACC_ITEM = U_OWN
DUMMY = U_OWN + I_OWN
ACC_ROWS = 25088
ZROWS = ACC_ROWS // NS

LOOK_I = 4
LOOK_G = 1
LAG_S = 2

U_RPT, U_LAST = 936, U_OWN - 15 * 936
I_RPT, I_LAST = 624, I_OWN - 15 * 624

_mesh = plsc.VectorSubcoreMesh(core_axis_name="c", subcore_axis_name="s")

_BCAST_DNUMS = lax.GatherDimensionNumbers(
    offset_dims=(), collapsed_slice_dims=(0,), start_index_map=(0,)
)


def _lane_bcast(v, lane_idx):
    idx = jnp.full((LANES, 1), lane_idx, jnp.int32)
    return lax.gather(
        v, idx, _BCAST_DNUMS, (1,), mode=lax.GatherScatterMode.PROMISE_IN_BOUNDS
    )
_params = pltpu.CompilerParams(needs_layout_passes=False, use_tc_tiling_on_sc=False)

_prop_scratch = (
    [
        pltpu.VMEM_SHARED((ACC_ROWS, D), jnp.float32),
        pltpu.VMEM((NBI, CHUNK), jnp.int32),
        pltpu.VMEM((NBI, CHUNK), jnp.float32),
    ]
    + [pltpu.VMEM((CHUNK,), jnp.int32) for _ in range(NBI)]
    + [pltpu.VMEM((CHUNK, D), jnp.float32) for _ in range(NBR)]
    + [pltpu.SemaphoreType.DMA for _ in range(NBI + 2 * NBR)]
)


@functools.partial(
    pl.kernel,
    mesh=_mesh,
    compiler_params=_params,
    out_type=jax.ShapeDtypeStruct((N_PAD, D), jnp.float32),
    scratch_types=_prop_scratch,
)
def _propagate(emb, srcp, dstp, wp, zrows, out, acc, src_r, w_r, *rest):
    dst_r = rest[0:NBI]
    rows = rest[NBI : NBI + NBR]
    sem_i = rest[NBI + NBR : 2 * NBI + NBR]
    sem_g = rest[2 * NBI + NBR : 2 * NBI + 2 * NBR]
    sem_s = rest[2 * NBI + 2 * NBR : 2 * NBI + 3 * NBR]

    c = lax.axis_index("c")
    s = lax.axis_index("s")
    ku = c * U_OWN
    ki = c * I_OWN
    edge_base = s * EDGES_PER_TILE

    pltpu.sync_copy(zrows, acc.at[pl.ds(s * ZROWS, ZROWS)])
    plsc.subcore_barrier()

    def fire_idx(q, slot):
        base = edge_base + q * CHUNK
        pltpu.async_copy(srcp.at[pl.ds(base, CHUNK)], src_r.at[slot], sem_i[slot])
        pltpu.async_copy(dstp.at[pl.ds(base, CHUNK)], dst_r[slot], sem_i[slot])
        pltpu.async_copy(wp.at[pl.ds(base, CHUNK)], w_r.at[slot], sem_i[slot])

    def wait_idx(q, slot):
        base = edge_base + q * CHUNK
        pltpu.make_async_copy(srcp.at[pl.ds(base, CHUNK)], src_r.at[slot], sem_i[slot]).wait()
        pltpu.make_async_copy(dstp.at[pl.ds(base, CHUNK)], dst_r[slot], sem_i[slot]).wait()
        pltpu.make_async_copy(wp.at[pl.ds(base, CHUNK)], w_r.at[slot], sem_i[slot]).wait()

    def fire_gather(islot, rslot):
        pltpu.async_copy(emb.at[src_r.at[islot]], rows[rslot], sem_g[rslot])

    def wait_gather(islot, rslot):
        pltpu.make_async_copy(emb.at[src_r.at[islot]], rows[rslot], sem_g[rslot]).wait()

    def fire_scatter(islot, rslot):
        pltpu.async_copy(rows[rslot], acc.at[dst_r[islot]], sem_s[rslot], add=True)

    def wait_scatter(islot, rslot):
        pltpu.make_async_copy(rows[rslot], acc.at[dst_r[islot]], sem_s[rslot]).wait()

    for q0 in range(LOOK_I):
        fire_idx(q0, q0)
    for q0 in range(LOOK_G):
        wait_idx(q0, q0)
        fire_gather(q0, q0)

    def outer(sg, carry):
        q0 = sg * NBI
        for u in range(NBI):
            q = q0 + u
            b = u % NBR

            @pl.when(q >= LAG_S)
            def _(u=u, b=b):
                wait_scatter((u - LAG_S) % NBI, (b - LAG_S) % NBR)

            @pl.when(q + LOOK_I < CHUNKS_PER_TILE)
            def _(q=q, u=u):
                fire_idx(q + LOOK_I, (u + LOOK_I) % NBI)

            @pl.when(q + LOOK_G < CHUNKS_PER_TILE)
            def _(q=q, u=u, b=b):
                wait_idx(q + LOOK_G, (u + LOOK_G) % NBI)
                fire_gather((u + LOOK_G) % NBI, (b + LOOK_G) % NBR)

            wait_gather(u, b)

            def grp(gg, carry2, u=u, b=b):
                o = gg * LANES
                d = dst_r[u][pl.ds(o, LANES)]
                isu = d < N_USERS
                lu = d - ku
                li = d - (N_USERS + ki) + ACC_ITEM
                loc = jnp.where(isu, lu, li)
                ok = jnp.where(
                    isu,
                    (lu >= 0) & (lu < U_OWN),
                    (li >= ACC_ITEM) & (li < DUMMY),
                )
                dummy = DUMMY + ((d + gg) & 63)
                dst_r[u][pl.ds(o, LANES)] = jnp.where(ok, loc, dummy)
                wv = w_r[u, pl.ds(o, LANES)]
                wes = [_lane_bcast(wv, e16) for e16 in range(LANES)]
                nj = D // LANES
                for blk in range(0, LANES, 4):
                    es = [o + blk + t for t in range(4)]
                    vals = [
                        rows[b][e, pl.ds(j * LANES, LANES)]
                        for e in es
                        for j in range(nj)
                    ]
                    scaled = [
                        vals[t * nj + j] * wes[blk + t]
                        for t in range(4)
                        for j in range(nj)
                    ]
                    for t in range(4):
                        for j in range(nj):
                            rows[b][es[t], pl.ds(j * LANES, LANES)] = scaled[t * nj + j]
                return carry2

            lax.fori_loop(0, CHUNK // LANES, grp, 0)
            fire_scatter(u, b)
        return carry

    lax.fori_loop(0, CHUNKS_PER_TILE // NBI, outer, 0)
    for qq in range(CHUNKS_PER_TILE - LAG_S, CHUNKS_PER_TILE):
        wait_scatter(qq % NBI, qq % NBR)

    plsc.subcore_barrier()

    @pl.when(s < 15)
    def _():
        pltpu.sync_copy(
            acc.at[pl.ds(s * U_RPT, U_RPT)], out.at[pl.ds(c * U_OWN + s * U_RPT, U_RPT)]
        )
        pltpu.sync_copy(
            acc.at[pl.ds(ACC_ITEM + s * I_RPT, I_RPT)],
            out.at[pl.ds(ITEM_BASE + c * I_OWN + s * I_RPT, I_RPT)],
        )

    @pl.when(s == 15)
    def _():
        pltpu.sync_copy(
            acc.at[pl.ds(15 * U_RPT, U_LAST)],
            out.at[pl.ds(c * U_OWN + 15 * U_RPT, U_LAST)],
        )
        pltpu.sync_copy(
            acc.at[pl.ds(ACC_ITEM + 15 * I_RPT, I_LAST)],
            out.at[pl.ds(ITEM_BASE + c * I_OWN + 15 * I_RPT, I_LAST)],
        )


def _make_forward(batch):
    bt = batch // (NC * NS)

    @functools.partial(
        pl.kernel,
        mesh=_mesh,
        compiler_params=_params,
        out_type=jax.ShapeDtypeStruct((batch,), jnp.float32),
        scratch_types=[
            pltpu.VMEM((bt,), jnp.int32),
            pltpu.VMEM((bt,), jnp.int32),
            pltpu.VMEM((bt, D), jnp.float32),
            pltpu.VMEM((bt, D), jnp.float32),
            pltpu.VMEM((bt, D), jnp.float32),
            pltpu.VMEM((bt,), jnp.float32),
            pltpu.SemaphoreType.DMA,
        ],
    )
    def _forward(e0, e1, e2, e3, users, items, gamma, uidx, iidx, rows, uacc, iacc, gam, sem):
        c = lax.axis_index("c")
        s = lax.axis_index("s")
        b0 = (s * NC + c) * bt
        pltpu.sync_copy(users.at[pl.ds(b0, bt)], uidx)
        pltpu.sync_copy(items.at[pl.ds(b0, bt)], iidx)

        def off_body(gg, carry):
            o = gg * LANES
            iidx[pl.ds(o, LANES)] = iidx[pl.ds(o, LANES)] + ITEM_BASE
            return carry

        lax.fori_loop(0, bt // LANES, off_body, 0)

        for idx, dacc in ((uidx, uacc), (iidx, iacc)):
            for t, tab in enumerate((e0, e1, e2, e3)):
                pltpu.async_copy(tab.at[idx], rows, sem).wait()

                def acc_body(e, carry, t=t, dacc=dacc):
                    for j in range(D // LANES):
                        sl = pl.ds(j * LANES, LANES)
                        v = rows[e, sl]
                        if t:
                            v = dacc[e, sl] + v
                        dacc[e, sl] = v
                    return carry

                lax.fori_loop(0, bt, acc_body, 0)

        scale = 1.0 / float((LAYERS + 1) ** 2)
        lane = lax.iota(jnp.int32, LANES)

        def dot_body(g, carry):
            o = g * LANES
            accv = jnp.zeros((LANES,), jnp.float32)
            for e16 in range(LANES):
                e = o + e16
                ps = jnp.zeros((LANES,), jnp.float32)
                for j in range(D // LANES):
                    sl = pl.ds(j * LANES, LANES)
                    ps = ps + uacc[e, sl] * iacc[e, sl]
                tot = jnp.sum(ps) * scale
                accv = jnp.where(lane == e16, tot, accv)
            gam[pl.ds(o, LANES)] = accv
            return carry

        lax.fori_loop(0, bt // LANES, dot_body, 0)
        pltpu.sync_copy(gam, gamma.at[pl.ds(b0, bt)])

    return _forward


def kernel(users, items, edge_index, edge_weight, user_table, item_table):
    emb0 = jnp.concatenate(
        [user_table, jnp.zeros((PAD_ROWS, D), jnp.float32), item_table], axis=0
    )
    src = edge_index[0]
    dst = edge_index[1]
    src = jnp.where(src >= N_USERS, src + PAD_ROWS, src)
    pad = E_PAD - E
    srcp = jnp.concatenate([src, jnp.zeros((pad,), jnp.int32)])
    dstp = jnp.concatenate([dst, jnp.arange(pad, dtype=jnp.int32) % N_USERS])
    wp = jnp.concatenate([edge_weight, jnp.zeros((pad,), jnp.float32)])
    zrows = jnp.zeros((ZROWS, D), jnp.float32)

    e0 = emb0
    e1 = _propagate(e0, srcp, dstp, wp, zrows)
    e2 = _propagate(e1, srcp, dstp, wp, zrows)
    e3 = _propagate(e2, srcp, dstp, wp, zrows)
    fwd = _make_forward(users.shape[0])
    return fwd(e0, e1, e2, e3, users, items)
